# baseline (speedup 1.0000x reference)
import functools

import jax
import jax.numpy as jnp
from jax import lax
from jax.experimental import pallas as pl
from jax.experimental.pallas import tpu as pltpu
from jax.experimental.pallas import tpu_sc as plsc

N = 20000
K = 1024
NPAD = 20480
SUB = 8
LANE = NPAD // SUB
ROWS = NPAD // 8
NEG_INF = float("-inf")

SC_CORES = 2
SC_SUBCORES = 16
SC_WORKERS = SC_CORES * SC_SUBCORES
CHUNK = NPAD // SC_WORKERS


def _fps_kernel(planes_ref, rows_ref, idx_ref, clusters_ref, grid_ref):
    idx_grid = (lax.broadcasted_iota(jnp.int32, (SUB, LANE), 0) * LANE
                + lax.broadcasted_iota(jnp.int32, (SUB, LANE), 1))
    grid_ref[...] = idx_grid
    valid = idx_grid < N

    def dist_to(row):
        d = (planes_ref[0] - row[0:1, 0:1]) ** 2
        for c in range(1, 6):
            d = d + (planes_ref[c] - row[0:1, c:c + 1]) ** 2
        return d

    idx_ref[0:1, :] = jnp.zeros((1, 1), jnp.int32)
    row0 = rows_ref[0, 0:1, :]
    clusters_ref[0:1, :] = row0
    dists0 = jnp.where(valid, dist_to(row0), NEG_INF)

    def body(t, dists):
        maxv = jnp.max(dists, axis=(0, 1), keepdims=True)
        cand = jnp.where(dists == maxv, grid_ref[...], NPAD)
        nxt11 = jnp.min(cand, axis=(0, 1), keepdims=True)
        idx_ref[pl.ds(t, 1), :] = nxt11
        nxt = nxt11[0, 0]
        row = rows_ref[nxt // 8, pl.ds(nxt % 8, 1), :]
        clusters_ref[pl.ds(t, 1), :] = row
        return jnp.minimum(dists, dist_to(row))

    lax.fori_loop(1, K, body, dists0)


def _fps_call(planes, rows):
    return pl.pallas_call(
        _fps_kernel,
        out_shape=(
            jax.ShapeDtypeStruct((K, 1), jnp.int32),
            jax.ShapeDtypeStruct((K, 8), jnp.float32),
        ),
        scratch_shapes=[pltpu.VMEM((SUB, LANE), jnp.int32)],
    )(planes, rows)


_KNN_B = 2048


def _knn_kernel(pts_ref, ct_ref, nn_ref, idxe_ref):
    p = pts_ref[...]
    ct = ct_ref[...]
    a2 = jnp.sum(p * p, axis=1, keepdims=True)
    b2 = jnp.sum(ct * ct, axis=0, keepdims=True)
    dots = jnp.dot(p, ct, preferred_element_type=jnp.float32)
    d2 = (a2 + b2) - 2.0 * dots
    m = jnp.min(d2, axis=1, keepdims=True)
    ks = lax.broadcasted_iota(jnp.int32, d2.shape, 1)
    nn = jnp.min(jnp.where(d2 == m, ks, K), axis=1, keepdims=True)
    nn_ref[...] = nn
    idxe_ref[...] = nn * 16 + lax.broadcasted_iota(
        jnp.int32, (nn.shape[0], 16), 1)


def _knn_call(pts, ct):
    grid = NPAD // _KNN_B
    return pl.pallas_call(
        _knn_kernel,
        grid=(grid,),
        in_specs=[
            pl.BlockSpec((_KNN_B, 8), lambda i: (i, 0)),
            pl.BlockSpec((8, K), lambda i: (0, 0)),
        ],
        out_specs=(
            pl.BlockSpec((_KNN_B, 1), lambda i: (i, 0)),
            pl.BlockSpec((_KNN_B, 16), lambda i: (i, 0)),
        ),
        out_shape=(
            jax.ShapeDtypeStruct((NPAD, 1), jnp.int32),
            jax.ShapeDtypeStruct((NPAD, 16), jnp.int32),
        ),
    )(pts, ct)


def _scatter_body(idxe_hbm, feats_hbm, zeros_hbm, out_hbm, idxe_v, rows_v,
                  table_v):
    c = lax.axis_index("c")
    s = lax.axis_index("s")
    wid = c * SC_SUBCORES + s
    base = wid * CHUNK * 16
    pltpu.sync_copy(idxe_hbm.at[pl.ds(base, CHUNK * 16)], idxe_v)
    pltpu.sync_copy(feats_hbm.at[pl.ds(base, CHUNK * 16)], rows_v)
    pltpu.sync_copy(zeros_hbm, table_v)

    def step(i, _):
        idx16 = idxe_v[pl.ds(i * 16, 16)]
        dat16 = rows_v[pl.ds(i * 16, 16)]
        plsc.addupdate_scatter(table_v, [idx16], dat16)
        return 0

    lax.fori_loop(0, CHUNK, step, 0)
    pltpu.sync_copy(table_v, out_hbm.at[wid])


def _scatter_call(idx_e, feats, zeros):
    mesh = plsc.VectorSubcoreMesh(core_axis_name="c", subcore_axis_name="s")
    fn = pl.kernel(
        _scatter_body,
        mesh=mesh,
        compiler_params=pltpu.CompilerParams(needs_layout_passes=False),
        out_type=jax.ShapeDtypeStruct((SC_WORKERS, K * 16), jnp.float32),
        scratch_types=[
            pltpu.VMEM((CHUNK * 16,), jnp.int32),
            pltpu.VMEM((CHUNK * 16,), jnp.float32),
            pltpu.VMEM((K * 16,), jnp.float32),
        ],
    )
    return fn(idx_e.reshape(-1), feats.reshape(-1), zeros)


def _finalize_kernel(parts_ref, out_ref):
    t = parts_ref[0]
    for w in range(1, SC_WORKERS):
        t = t + parts_ref[w]
    cnt = jnp.maximum(t[:, 6:7], 1.0)
    out_ref[...] = t / cnt


def _finalize_call(parts):
    return pl.pallas_call(
        _finalize_kernel,
        out_shape=jax.ShapeDtypeStruct((K, 16), jnp.float32),
    )(parts)


def kernel(x, pos, batch):
    pos6d = jnp.concatenate([pos, x], axis=1)
    pts = jnp.pad(pos6d, ((0, NPAD - N), (0, 2)))
    planes = pts.T[:6].reshape(6, SUB, LANE)
    rows = pts.reshape(ROWS, 8, 8)

    idx2, clusters = _fps_call(planes, rows)
    idx = idx2[:, 0]

    nn2, idx_e = _knn_call(pts, clusters.T)
    nn_full = nn2[:, 0]

    feats = jnp.concatenate(
        [x, pos, jnp.ones((N, 1), jnp.float32)], axis=1)
    feats = jnp.pad(feats, ((0, NPAD - N), (0, 9)))
    parts = _scatter_call(idx_e, feats,
                          jnp.zeros((K * 16,), jnp.float32))
    pooled = _finalize_call(parts.reshape(SC_WORKERS, K, 16))

    x_new = pooled[:, 0:3]
    pos_new = pooled[:, 3:6]
    nn = nn_full[:N]
    edge_index = jnp.stack([jnp.arange(N, dtype=jnp.int32), nn], axis=0)
    batch_new = jnp.take(batch, idx, axis=0)
    return (x_new, pos_new, batch_new, edge_index)


# speedup vs baseline: 1.0803x; 1.0803x over previous
import functools

import jax
import jax.numpy as jnp
from jax import lax
from jax.experimental import pallas as pl
from jax.experimental.pallas import tpu as pltpu
from jax.experimental.pallas import tpu_sc as plsc

N = 20000
K = 1024
NPAD = 20480
SUB = 8
LANE = NPAD // SUB
ROWS = NPAD // 8
NEG_INF = float("-inf")

SC_CORES = 2
SC_SUBCORES = 16
SC_WORKERS = SC_CORES * SC_SUBCORES
CHUNK = NPAD // SC_WORKERS


def _fps_kernel(planes_ref, rows_ref, idx_ref, clusters_ref, grid_ref):
    idx_grid = (lax.broadcasted_iota(jnp.int32, (SUB, LANE), 0) * LANE
                + lax.broadcasted_iota(jnp.int32, (SUB, LANE), 1))
    grid_ref[...] = idx_grid.astype(jnp.float32)
    valid = idx_grid < N

    def dist_to(row):
        d = (planes_ref[0] - row[0:1, 0:1]) ** 2
        for c in range(1, 6):
            d = d + (planes_ref[c] - row[0:1, c:c + 1]) ** 2
        return d

    idx_ref[0:1, :] = jnp.zeros((1, 1), jnp.int32)
    row0 = rows_ref[0, 0:1, :]
    clusters_ref[0:1, :] = row0
    dists0 = jnp.where(valid, dist_to(row0), NEG_INF)

    def body(t, dists):
        maxv = jnp.max(dists, axis=(0, 1), keepdims=True)
        cand = jnp.where(dists == maxv, grid_ref[...], jnp.inf)
        nxt11 = jnp.min(cand, axis=(0, 1), keepdims=True).astype(jnp.int32)
        idx_ref[pl.ds(t, 1), :] = nxt11
        nxt = nxt11[0, 0]
        row = rows_ref[nxt // 8, pl.ds(nxt % 8, 1), :]
        clusters_ref[pl.ds(t, 1), :] = row
        return jnp.minimum(dists, dist_to(row))

    lax.fori_loop(1, K, body, dists0)


def _fps_call(planes, rows):
    return pl.pallas_call(
        _fps_kernel,
        out_shape=(
            jax.ShapeDtypeStruct((K, 1), jnp.int32),
            jax.ShapeDtypeStruct((K, 8), jnp.float32),
        ),
        scratch_shapes=[pltpu.VMEM((SUB, LANE), jnp.float32)],
    )(planes, rows)


_KNN_B = 2048


def _knn_kernel(pts_ref, ct_ref, nn_ref, idxe_ref):
    p = pts_ref[...]
    ct = ct_ref[...]
    a2 = jnp.sum(p * p, axis=1, keepdims=True)
    b2 = jnp.sum(ct * ct, axis=0, keepdims=True)
    dots = jnp.dot(p, ct, preferred_element_type=jnp.float32)
    d2 = (a2 + b2) - 2.0 * dots
    m = jnp.min(d2, axis=1, keepdims=True)
    ks = lax.broadcasted_iota(jnp.int32, d2.shape, 1)
    nn = jnp.min(jnp.where(d2 == m, ks, K), axis=1, keepdims=True)
    nn_ref[...] = nn
    idxe_ref[...] = nn * 16 + lax.broadcasted_iota(
        jnp.int32, (nn.shape[0], 16), 1)


def _knn_call(pts, ct):
    grid = NPAD // _KNN_B
    return pl.pallas_call(
        _knn_kernel,
        grid=(grid,),
        in_specs=[
            pl.BlockSpec((_KNN_B, 8), lambda i: (i, 0)),
            pl.BlockSpec((8, K), lambda i: (0, 0)),
        ],
        out_specs=(
            pl.BlockSpec((_KNN_B, 1), lambda i: (i, 0)),
            pl.BlockSpec((_KNN_B, 16), lambda i: (i, 0)),
        ),
        out_shape=(
            jax.ShapeDtypeStruct((NPAD, 1), jnp.int32),
            jax.ShapeDtypeStruct((NPAD, 16), jnp.int32),
        ),
    )(pts, ct)


def _scatter_body(idxe_hbm, feats_hbm, zeros_hbm, out_hbm, idxe_v, rows_v,
                  table_v):
    c = lax.axis_index("c")
    s = lax.axis_index("s")
    wid = c * SC_SUBCORES + s
    base = wid * CHUNK * 16
    pltpu.sync_copy(idxe_hbm.at[pl.ds(base, CHUNK * 16)], idxe_v)
    pltpu.sync_copy(feats_hbm.at[pl.ds(base, CHUNK * 16)], rows_v)
    pltpu.sync_copy(zeros_hbm, table_v)

    def step(i, _):
        idx16 = idxe_v[pl.ds(i * 16, 16)]
        dat16 = rows_v[pl.ds(i * 16, 16)]
        plsc.addupdate_scatter(table_v, [idx16], dat16)
        return 0

    lax.fori_loop(0, CHUNK, step, 0)
    pltpu.sync_copy(table_v, out_hbm.at[wid])


def _scatter_call(idx_e, feats, zeros):
    mesh = plsc.VectorSubcoreMesh(core_axis_name="c", subcore_axis_name="s")
    fn = pl.kernel(
        _scatter_body,
        mesh=mesh,
        compiler_params=pltpu.CompilerParams(needs_layout_passes=False),
        out_type=jax.ShapeDtypeStruct((SC_WORKERS, K * 16), jnp.float32),
        scratch_types=[
            pltpu.VMEM((CHUNK * 16,), jnp.int32),
            pltpu.VMEM((CHUNK * 16,), jnp.float32),
            pltpu.VMEM((K * 16,), jnp.float32),
        ],
    )
    return fn(idx_e.reshape(-1), feats.reshape(-1), zeros)


def _finalize_kernel(parts_ref, out_ref):
    t = parts_ref[0]
    for w in range(1, SC_WORKERS):
        t = t + parts_ref[w]
    cnt = jnp.maximum(t[:, 6:7], 1.0)
    out_ref[...] = t / cnt


def _finalize_call(parts):
    return pl.pallas_call(
        _finalize_kernel,
        out_shape=jax.ShapeDtypeStruct((K, 16), jnp.float32),
    )(parts)


def kernel(x, pos, batch):
    pos6d = jnp.concatenate([pos, x], axis=1)
    pts = jnp.pad(pos6d, ((0, NPAD - N), (0, 2)))
    planes = pts.T[:6].reshape(6, SUB, LANE)
    rows = pts.reshape(ROWS, 8, 8)

    idx2, clusters = _fps_call(planes, rows)
    idx = idx2[:, 0]

    nn2, idx_e = _knn_call(pts, clusters.T)
    nn_full = nn2[:, 0]

    feats = jnp.concatenate(
        [x, pos, jnp.ones((N, 1), jnp.float32)], axis=1)
    feats = jnp.pad(feats, ((0, NPAD - N), (0, 9)))
    parts = _scatter_call(idx_e, feats,
                          jnp.zeros((K * 16,), jnp.float32))
    pooled = _finalize_call(parts.reshape(SC_WORKERS, K, 16))

    x_new = pooled[:, 0:3]
    pos_new = pooled[:, 3:6]
    nn = nn_full[:N]
    edge_index = jnp.stack([jnp.arange(N, dtype=jnp.int32), nn], axis=0)
    batch_new = jnp.take(batch, idx, axis=0)
    return (x_new, pos_new, batch_new, edge_index)


# speedup vs baseline: 1.1326x; 1.0484x over previous
import functools

import jax
import jax.numpy as jnp
from jax import lax
from jax.experimental import pallas as pl
from jax.experimental.pallas import tpu as pltpu
from jax.experimental.pallas import tpu_sc as plsc

N = 20000
K = 1024
NPAD = 20480
SUB = 8
LANE = NPAD // SUB
ROWS = NPAD // 8
NEG_INF = float("-inf")

SC_CORES = 2
SC_SUBCORES = 16
SC_WORKERS = SC_CORES * SC_SUBCORES
CHUNK = NPAD // SC_WORKERS


_T = NPAD // 1024


def _sublane_bcast(v, op):
    for k in (4, 2, 1):
        v = op(v, pltpu.roll(v, k, axis=0))
    return v


def _fps_kernel(planes_ref, rows_ref, idx_ref, clusters_ref, grid_ref):
    idx_grid = (lax.broadcasted_iota(jnp.int32, (_T, 8, 128), 0) * 1024
                + lax.broadcasted_iota(jnp.int32, (_T, 8, 128), 1) * 128
                + lax.broadcasted_iota(jnp.int32, (_T, 8, 128), 2))
    grid_ref[...] = idx_grid.astype(jnp.float32)
    valid = idx_grid < N

    def dist_to(row):
        d = (planes_ref[0] - row[0:1, 0:1]) ** 2
        for c in range(1, 6):
            d = d + (planes_ref[c] - row[0:1, c:c + 1]) ** 2
        return d

    idx_ref[0:1, :] = jnp.zeros((1, 1), jnp.int32)
    row0 = rows_ref[0, 0:1, :]
    clusters_ref[0:1, :] = row0
    dists0 = jnp.where(valid, dist_to(row0), NEG_INF)

    def body(t, dists):
        m = jnp.max(jnp.max(dists, axis=0), axis=(0, 1), keepdims=True)
        cand = jnp.where(dists == m, grid_ref[...], jnp.inf)
        nf = jnp.min(jnp.min(cand, axis=0), axis=(0, 1), keepdims=True)
        idx_ref[pl.ds(t, 1), :] = nf.astype(jnp.int32)
        nxt = nf[0, 0].astype(jnp.int32)
        row = rows_ref[nxt // 8, pl.ds(nxt % 8, 1), :]
        clusters_ref[pl.ds(t, 1), :] = row
        return jnp.minimum(dists, dist_to(row))

    lax.fori_loop(1, K, body, dists0)


def _fps_call(planes, rows):
    return pl.pallas_call(
        _fps_kernel,
        out_shape=(
            jax.ShapeDtypeStruct((K, 1), jnp.int32),
            jax.ShapeDtypeStruct((K, 8), jnp.float32),
        ),
        scratch_shapes=[pltpu.VMEM((_T, 8, 128), jnp.float32)],
    )(planes, rows)


_KNN_B = 2048


def _knn_kernel(pts_ref, ct_ref, nn_ref, idxe_ref):
    p = pts_ref[...]
    ct = ct_ref[...]
    a2 = jnp.sum(p * p, axis=1, keepdims=True)
    b2 = jnp.sum(ct * ct, axis=0, keepdims=True)
    dots = jnp.dot(p, ct, preferred_element_type=jnp.float32)
    d2 = (a2 + b2) - 2.0 * dots
    m = jnp.min(d2, axis=1, keepdims=True)
    ks = lax.broadcasted_iota(jnp.int32, d2.shape, 1)
    nn = jnp.min(jnp.where(d2 == m, ks, K), axis=1, keepdims=True)
    nn_ref[...] = nn
    idxe_ref[...] = nn * 16 + lax.broadcasted_iota(
        jnp.int32, (nn.shape[0], 16), 1)


def _knn_call(pts, ct):
    grid = NPAD // _KNN_B
    return pl.pallas_call(
        _knn_kernel,
        grid=(grid,),
        in_specs=[
            pl.BlockSpec((_KNN_B, 8), lambda i: (i, 0)),
            pl.BlockSpec((8, K), lambda i: (0, 0)),
        ],
        out_specs=(
            pl.BlockSpec((_KNN_B, 1), lambda i: (i, 0)),
            pl.BlockSpec((_KNN_B, 16), lambda i: (i, 0)),
        ),
        out_shape=(
            jax.ShapeDtypeStruct((NPAD, 1), jnp.int32),
            jax.ShapeDtypeStruct((NPAD, 16), jnp.int32),
        ),
    )(pts, ct)


def _scatter_body(idxe_hbm, feats_hbm, zeros_hbm, out_hbm, idxe_v, rows_v,
                  table_v):
    c = lax.axis_index("c")
    s = lax.axis_index("s")
    wid = c * SC_SUBCORES + s
    base = wid * CHUNK * 16
    pltpu.sync_copy(idxe_hbm.at[pl.ds(base, CHUNK * 16)], idxe_v)
    pltpu.sync_copy(feats_hbm.at[pl.ds(base, CHUNK * 16)], rows_v)
    pltpu.sync_copy(zeros_hbm, table_v)

    def step(i, _):
        idx16 = idxe_v[pl.ds(i * 16, 16)]
        dat16 = rows_v[pl.ds(i * 16, 16)]
        plsc.addupdate_scatter(table_v, [idx16], dat16)
        return 0

    lax.fori_loop(0, CHUNK, step, 0)
    pltpu.sync_copy(table_v, out_hbm.at[wid])


def _scatter_call(idx_e, feats, zeros):
    mesh = plsc.VectorSubcoreMesh(core_axis_name="c", subcore_axis_name="s")
    fn = pl.kernel(
        _scatter_body,
        mesh=mesh,
        compiler_params=pltpu.CompilerParams(needs_layout_passes=False),
        out_type=jax.ShapeDtypeStruct((SC_WORKERS, K * 16), jnp.float32),
        scratch_types=[
            pltpu.VMEM((CHUNK * 16,), jnp.int32),
            pltpu.VMEM((CHUNK * 16,), jnp.float32),
            pltpu.VMEM((K * 16,), jnp.float32),
        ],
    )
    return fn(idx_e.reshape(-1), feats.reshape(-1), zeros)


def _finalize_kernel(parts_ref, out_ref):
    t = parts_ref[0]
    for w in range(1, SC_WORKERS):
        t = t + parts_ref[w]
    cnt = jnp.maximum(t[:, 6:7], 1.0)
    out_ref[...] = t / cnt


def _finalize_call(parts):
    return pl.pallas_call(
        _finalize_kernel,
        out_shape=jax.ShapeDtypeStruct((K, 16), jnp.float32),
    )(parts)


def kernel(x, pos, batch):
    pos6d = jnp.concatenate([pos, x], axis=1)
    pts = jnp.pad(pos6d, ((0, NPAD - N), (0, 2)))
    planes = pts.T[:6].reshape(6, NPAD // 1024, 8, 128)
    rows = pts.reshape(ROWS, 8, 8)

    idx2, clusters = _fps_call(planes, rows)
    idx = idx2[:, 0]

    nn2, idx_e = _knn_call(pts, clusters.T)
    nn_full = nn2[:, 0]

    feats = jnp.concatenate(
        [x, pos, jnp.ones((N, 1), jnp.float32)], axis=1)
    feats = jnp.pad(feats, ((0, NPAD - N), (0, 9)))
    parts = _scatter_call(idx_e, feats,
                          jnp.zeros((K * 16,), jnp.float32))
    pooled = _finalize_call(parts.reshape(SC_WORKERS, K, 16))

    x_new = pooled[:, 0:3]
    pos_new = pooled[:, 3:6]
    nn = nn_full[:N]
    edge_index = jnp.stack([jnp.arange(N, dtype=jnp.int32), nn], axis=0)
    batch_new = jnp.take(batch, idx, axis=0)
    return (x_new, pos_new, batch_new, edge_index)
